# Initial kernel scaffold; baseline (speedup 1.0000x reference)
import jax
import jax.numpy as jnp
from jax.experimental import pallas as pl


def kernel(inputs, gate_w, w1, w2):
    raise NotImplementedError("write your pallas kernel here")


# speedup vs baseline: 2.2949x; 2.2949x over previous
import functools

import jax
import jax.numpy as jnp
from jax.experimental import pallas as pl
from jax.experimental.pallas import tpu as pltpu

N_EXP = 16
DM = 768
DF = 768
TOKENS = 2048


def _moe_dense_kernel(x_ref, gw_ref, w1_ref, w2_ref, out_ref, wtok_ref):
    e = pl.program_id(0)

    @pl.when(e == 0)
    def _():
        x = x_ref[...]
        logits = jnp.dot(x, gw_ref[...], preferred_element_type=jnp.float32)
        m1 = jnp.max(logits, axis=1, keepdims=True)
        lanes = jax.lax.broadcasted_iota(jnp.int32, logits.shape, 1)
        is_m1 = logits == m1
        a1 = jnp.min(jnp.where(is_m1, lanes, N_EXP), axis=1, keepdims=True)
        masked = jnp.where(lanes == a1, -jnp.inf, logits)
        m2 = jnp.max(masked, axis=1, keepdims=True)
        is_m2 = masked == m2
        a2 = jnp.min(jnp.where(is_m2, lanes, N_EXP), axis=1, keepdims=True)
        p1 = 1.0 / (1.0 + jnp.exp(m2 - m1))
        p2 = 1.0 - p1
        wtok_ref[...] = jnp.where(lanes == a1, p1, 0.0) + jnp.where(
            lanes == a2, p2, 0.0
        )
        out_ref[...] = jnp.zeros_like(out_ref)

    x = x_ref[...]
    h = jnp.dot(x, w1_ref[0], preferred_element_type=jnp.float32)
    h = h * jax.nn.sigmoid(h)
    y = jnp.dot(h, w2_ref[0], preferred_element_type=jnp.float32)
    wtok = wtok_ref[...]
    lanes2 = jax.lax.broadcasted_iota(jnp.int32, wtok.shape, 1)
    w_col = jnp.sum(jnp.where(lanes2 == e, wtok, 0.0), axis=1, keepdims=True)
    out_ref[...] += w_col * y


def kernel(inputs, gate_w, w1, w2):
    x = inputs.reshape(-1, inputs.shape[-1])
    out = pl.pallas_call(
        _moe_dense_kernel,
        grid=(N_EXP,),
        in_specs=[
            pl.BlockSpec((TOKENS, DM), lambda e: (0, 0)),
            pl.BlockSpec((DM, N_EXP), lambda e: (0, 0)),
            pl.BlockSpec((1, DM, DF), lambda e: (e, 0, 0)),
            pl.BlockSpec((1, DF, DM), lambda e: (e, 0, 0)),
        ],
        out_specs=pl.BlockSpec((TOKENS, DM), lambda e: (0, 0)),
        out_shape=jax.ShapeDtypeStruct((TOKENS, DM), jnp.float32),
        scratch_shapes=[pltpu.VMEM((TOKENS, N_EXP), jnp.float32)],
    )(x, gate_w, w1, w2)
    return out.reshape(inputs.shape)
